# baseline (speedup 1.0000x reference)
import jax
import jax.numpy as jnp
from jax import lax
from jax.experimental import pallas as pl
from jax.experimental.pallas import tpu as pltpu
from jax.experimental.pallas import tpu_sc as plsc

_V = 8192
_NC, _NS = 2, 16
_NW = _NC * _NS
_TPW = 256
_GS = 16
_NG = _TPW // _GS


def _gather_body(x2_hbm, ycol_hbm, table_hbm, out_hbm, val_out_hbm,
                 idx_v, ycol_v, win_v, accv, sem_r, sem_w):
    wid = lax.axis_index("s") * _NC + lax.axis_index("c")
    base = wid * _TPW
    pltpu.sync_copy(x2_hbm.at[wid], idx_v)
    pltpu.sync_copy(ycol_hbm.at[wid], ycol_v)
    lane = lax.iota(jnp.int32, 16)

    def issue(q):
        xv = idx_v[q]
        yv = ycol_v[q]
        for l in range(_GS):
            xi = xv[l]
            t = base + q * _GS + l
            pltpu.async_copy(table_hbm.at[xi], out_hbm.at[t], sem_r)
            st = pl.multiple_of(yv[l] & ~15, 16)
            pltpu.async_copy(table_hbm.at[xi, pl.ds(st, 16)],
                             win_v.at[q * _GS + l], sem_w)

    def drain():
        for l in range(_GS):
            pltpu.make_async_copy(table_hbm.at[0], out_hbm.at[base],
                                  sem_r).wait()
            pltpu.make_async_copy(table_hbm.at[0, pl.ds(0, 16)],
                                  win_v.at[0], sem_w).wait()

    def body(q, c):
        @pl.when(q < _NG)
        def _i():
            issue(q)

        @pl.when(q > 0)
        def _d():
            drain()
        return c

    lax.fori_loop(0, _NG + 1, body, 0)

    def extract(q, acc):
        yv = ycol_v[q]
        for l in range(_GS):
            w = win_v[q * _GS + l]
            acc = acc + jnp.where(lane == (yv[l] & 15), w, 0.0)
        return acc

    acc = lax.fori_loop(0, _NG, extract, jnp.zeros((16,), jnp.float32))
    accv[...] = acc
    pltpu.sync_copy(accv, val_out_hbm.at[wid])


def _lse_kernel(t_ref, lse_ref):
    t = t_ref[...]
    m = jnp.max(t, axis=1, keepdims=True)
    s = jnp.sum(jnp.exp(t - m), axis=1, keepdims=True)
    lse_ref[...] = m + jnp.log(s)


def _lse_gather_body(x3_hbm, lse_hbm, out_hbm, xv, lv, accv, sem):
    wid = lax.axis_index("s") * _NC + lax.axis_index("c")
    pltpu.sync_copy(x3_hbm.at[wid], xv)
    for r in range(2):
        pltpu.async_copy(lse_hbm.at[xv.at[r]], lv.at[r], sem)
        pltpu.make_async_copy(lse_hbm.at[xv.at[r]], lv.at[r], sem).wait()
    acc = jnp.zeros((16,), jnp.float32)
    for r in range(2):
        for j in range(8):
            acc = acc + lv[r, pl.ds(j * 16, 16)]
    accv[...] = acc
    pltpu.sync_copy(accv, out_hbm.at[wid])


def _loss_kernel(pl_ref, pv_ref, loss_ref):
    tot = jnp.sum(pl_ref[...]) - jnp.sum(pv_ref[...])
    loss_ref[...] = jnp.full((1, 1), tot / (_NW * _TPW), jnp.float32)


def kernel(x, y, table):
    xf = x.reshape(-1).astype(jnp.int32)
    yf = y.reshape(-1).astype(jnp.int32)
    ntok = xf.shape[0]
    mesh = plsc.VectorSubcoreMesh(core_axis_name="c", subcore_axis_name="s")

    _RB = 256
    lse = pl.pallas_call(
        _lse_kernel,
        grid=(_V // _RB,),
        in_specs=[pl.BlockSpec((_RB, _V), lambda i: (i, 0))],
        out_specs=pl.BlockSpec((_RB, 1), lambda i: (i, 0)),
        out_shape=jax.ShapeDtypeStruct((_V, 1), jnp.float32),
    )(table)

    x2 = xf.reshape(_NW, _NG, _GS)
    ycol = yf.reshape(_NW, _NG, _GS)
    gather = pl.kernel(
        _gather_body,
        out_type=[
            jax.ShapeDtypeStruct((ntok, _V), jnp.float32),
            jax.ShapeDtypeStruct((_NW, 16), jnp.float32),
        ],
        mesh=mesh,
        scratch_types=[
            pltpu.VMEM((_NG, _GS), jnp.int32),
            pltpu.VMEM((_NG, _GS), jnp.int32),
            pltpu.VMEM((_TPW, 16), jnp.float32),
            pltpu.VMEM((16,), jnp.float32),
            pltpu.SemaphoreType.DMA,
            pltpu.SemaphoreType.DMA,
        ],
    )
    logits, val_parts = gather(x2, ycol, table)

    x3 = xf.reshape(_NW, 2, 128)
    lse_gather = pl.kernel(
        _lse_gather_body,
        out_type=jax.ShapeDtypeStruct((_NW, 16), jnp.float32),
        mesh=mesh,
        scratch_types=[
            pltpu.VMEM((2, 128), jnp.int32),
            pltpu.VMEM((2, 128), jnp.float32),
            pltpu.VMEM((16,), jnp.float32),
            pltpu.SemaphoreType.DMA,
        ],
    )
    lse_parts = lse_gather(x3, lse.reshape(_V))

    loss = pl.pallas_call(
        _loss_kernel,
        out_shape=jax.ShapeDtypeStruct((1, 1), jnp.float32),
    )(lse_parts, val_parts)

    return (logits, loss[0, 0])


# speedup vs baseline: 27.2048x; 27.2048x over previous
import jax
import jax.numpy as jnp
from jax import lax
from jax.experimental import pallas as pl
from jax.experimental.pallas import tpu as pltpu
from jax.experimental.pallas import tpu_sc as plsc

_V = 8192
_NC, _NS = 2, 16
_NW = _NC * _NS
_TPW = 256
_CH = 4
_NCHUNK = _TPW // _CH


def _gather_body(x2_hbm, ycol_hbm, table_hbm, out_hbm, val_out_hbm,
                 idx_v, ycol_v, buf0, buf1, accv, si0, si1, so0, so1):
    wid = lax.axis_index("s") * _NC + lax.axis_index("c")
    base = wid * _TPW
    pltpu.sync_copy(x2_hbm.at[wid], idx_v)
    pltpu.sync_copy(ycol_hbm.at[wid], ycol_v)
    lane = lax.iota(jnp.int32, 16)

    pltpu.async_copy(table_hbm.at[idx_v.at[0]], buf0, si0)
    pltpu.async_copy(table_hbm.at[idx_v.at[1]], buf1, si1)

    def body(o, acc):
        for b, (buf, si, so) in enumerate(((buf0, si0, so0),
                                           (buf1, si1, so1))):
            g = o * 2 + b
            dst = out_hbm.at[pl.ds(base + g * _CH, _CH)]
            pltpu.make_async_copy(table_hbm.at[idx_v.at[g]], buf, si).wait()
            pltpu.async_copy(buf, dst, so)
            yrow = ycol_v[g]
            for r in range(_CH):
                yi = yrow[r]
                st = pl.multiple_of(yi & ~15, 16)
                w = buf[r, pl.ds(st, 16)]
                acc = acc + jnp.where(lane == (yi & 15), w, 0.0)
            pltpu.make_async_copy(buf, dst, so).wait()

            @pl.when(g + 2 < _NCHUNK)
            def _next():
                pltpu.async_copy(table_hbm.at[idx_v.at[g + 2]], buf, si)
        return acc

    acc = lax.fori_loop(0, _NCHUNK // 2, body, jnp.zeros((16,), jnp.float32))
    accv[...] = acc
    pltpu.sync_copy(accv, val_out_hbm.at[wid])


def _lse_kernel(t_ref, lse_ref):
    t = t_ref[...]
    m = jnp.max(t, axis=1, keepdims=True)
    s = jnp.sum(jnp.exp(t - m), axis=1, keepdims=True)
    lse_ref[...] = m + jnp.log(s)


def _lse_gather_body(x3_hbm, lse_hbm, out_hbm, xv, lv, accv, sem):
    wid = lax.axis_index("s") * _NC + lax.axis_index("c")
    pltpu.sync_copy(x3_hbm.at[wid], xv)
    for r in range(2):
        pltpu.async_copy(lse_hbm.at[xv.at[r]], lv.at[r], sem)
        pltpu.make_async_copy(lse_hbm.at[xv.at[r]], lv.at[r], sem).wait()
    acc = jnp.zeros((16,), jnp.float32)
    for r in range(2):
        for j in range(8):
            acc = acc + lv[r, pl.ds(j * 16, 16)]
    accv[...] = acc
    pltpu.sync_copy(accv, out_hbm.at[wid])


def _loss_kernel(pl_ref, pv_ref, loss_ref):
    tot = jnp.sum(pl_ref[...]) - jnp.sum(pv_ref[...])
    loss_ref[...] = jnp.full((1, 1), tot / (_NW * _TPW), jnp.float32)


def kernel(x, y, table):
    xf = x.reshape(-1).astype(jnp.int32)
    yf = y.reshape(-1).astype(jnp.int32)
    ntok = xf.shape[0]
    mesh = plsc.VectorSubcoreMesh(core_axis_name="c", subcore_axis_name="s")

    _RB = 512
    lse = pl.pallas_call(
        _lse_kernel,
        grid=(_V // _RB,),
        in_specs=[pl.BlockSpec((_RB, _V), lambda i: (i, 0))],
        out_specs=pl.BlockSpec((_RB, 1), lambda i: (i, 0)),
        out_shape=jax.ShapeDtypeStruct((_V, 1), jnp.float32),
    )(table)

    x2 = xf.reshape(_NW, _NCHUNK, _CH)
    ycol = jnp.tile(yf.reshape(_NW, _NCHUNK, _CH), (1, 1, 16 // _CH))
    gather = pl.kernel(
        _gather_body,
        out_type=[
            jax.ShapeDtypeStruct((ntok, _V), jnp.float32),
            jax.ShapeDtypeStruct((_NW, 16), jnp.float32),
        ],
        mesh=mesh,
        scratch_types=[
            pltpu.VMEM((_NCHUNK, _CH), jnp.int32),
            pltpu.VMEM((_NCHUNK, 16), jnp.int32),
            pltpu.VMEM((_CH, _V), jnp.float32),
            pltpu.VMEM((_CH, _V), jnp.float32),
            pltpu.VMEM((16,), jnp.float32),
            pltpu.SemaphoreType.DMA,
            pltpu.SemaphoreType.DMA,
            pltpu.SemaphoreType.DMA,
            pltpu.SemaphoreType.DMA,
        ],
    )
    logits, val_parts = gather(x2, ycol, table)

    x3 = xf.reshape(_NW, 2, 128)
    lse_gather = pl.kernel(
        _lse_gather_body,
        out_type=jax.ShapeDtypeStruct((_NW, 16), jnp.float32),
        mesh=mesh,
        scratch_types=[
            pltpu.VMEM((2, 128), jnp.int32),
            pltpu.VMEM((2, 128), jnp.float32),
            pltpu.VMEM((16,), jnp.float32),
            pltpu.SemaphoreType.DMA,
        ],
    )
    lse_parts = lse_gather(x3, lse.reshape(_V))

    loss = pl.pallas_call(
        _loss_kernel,
        out_shape=jax.ShapeDtypeStruct((1, 1), jnp.float32),
    )(lse_parts, val_parts)

    return (logits, loss[0, 0])
